# baseline (speedup 1.0000x reference)
import math

import jax
import jax.numpy as jnp
import numpy as np
from jax import lax
from jax.experimental import pallas as pl
from jax.experimental.pallas import tpu as pltpu

_NBINS = 9
_POOL = 8
_GW = 16

_W1D = np.exp(-0.5 * ((np.arange(_GW, dtype=np.float64) - (_GW - 1) / 2.0) / (_GW // 2)) ** 2)
_GNORM = float(1.0 / (_W1D.sum() ** 2))
_COTS = [float(1.0 / np.tan(k * np.pi / _NBINS)) for k in range(1, _NBINS)]


def _smooth_matrix(n: int) -> np.ndarray:
    m = np.zeros((n, n), dtype=np.float64)
    for i in range(n):
        for d in (i - 1, i, i + 1):
            src = abs(d) if d < 0 else (2 * n - 2 - d if d >= n else d)
            m[i, src] += 2.0 if d == i else 1.0
    return m


def _hog_body(x_ref, kr_ref, kc_ref, p_ref, o_ref):
    for b in range(x_ref.shape[0]):
        _hog_one(x_ref, kr_ref, kc_ref, p_ref, o_ref, b)


def _hog_one(x_ref, kr_ref, kc_ref, p_ref, o_ref, b):
    Xb = x_ref[b, 0].astype(jnp.bfloat16)
    F, T = Xb.shape
    nF = F // _POOL
    nT = T // _POOL

    sv = lax.dot_general(kr_ref[...], Xb, (((1,), (0,)), ((), ())),
                         preferred_element_type=jnp.float32)
    sh = lax.dot_general(Xb, kc_ref[...], (((1,), (0,)), ((), ())),
                         preferred_element_type=jnp.float32)

    svl = jnp.concatenate([sv[:, 1:2], sv[:, :-1]], axis=1)
    svr = jnp.concatenate([sv[:, 1:], sv[:, T - 2:T - 1]], axis=1)
    gx = svl - svr
    shu = jnp.concatenate([sh[1:2, :], sh[:-1, :]], axis=0)
    shd = jnp.concatenate([sh[1:, :], sh[F - 2:F - 1, :]], axis=0)
    gy = shu - shd

    mag = jnp.sqrt(gx * gx + gy * gy)

    fi = lax.broadcasted_iota(jnp.int32, (F, 1), 0) % _GW
    fj = lax.broadcasted_iota(jnp.int32, (1, T), 1) % _GW
    half = jnp.float32((_GW - 1) / 2.0)
    inv_std = jnp.float32(1.0 / (_GW // 2))
    wi = jnp.exp(-0.5 * ((fi.astype(jnp.float32) - half) * inv_std) ** 2)
    wj = jnp.exp(-0.5 * ((fj.astype(jnp.float32) - half) * inv_std) ** 2)
    wmag = mag * (wi * jnp.float32(_GNORM)) * wj

    u = jnp.abs(gx)
    v = jnp.where(gx < 0, -gy, gy)
    r = jnp.where(gx == 0.0, jnp.float32(np.inf), v / u)

    wmag_bf = wmag.astype(jnp.bfloat16)
    P = p_ref[...]
    pooled = []
    for k in range(_NBINS):
        Sk = wmag_bf if k == 0 else jnp.where(
            r <= jnp.float32(_COTS[k - 1]), wmag_bf, jnp.bfloat16(0.0))
        qk = lax.dot_general(Sk, P, (((1,), (0,)), ((), ())),
                             preferred_element_type=jnp.float32)
        pooled.append(qk.reshape(nF, _POOL, nT).sum(axis=1))

    hist = [pooled[k] - pooled[k + 1] for k in range(_NBINS - 1)] + [pooled[_NBINS - 1]]
    ss = hist[0] * hist[0]
    for k in range(1, _NBINS):
        ss = ss + hist[k] * hist[k]
    inv = 1.0 / jnp.maximum(jnp.sqrt(ss), 1e-12)
    for k in range(_NBINS):
        o_ref[b, k] = hist[k] * inv


def kernel(x):
    B, C, F, T = x.shape
    nF = F // _POOL
    nT = T // _POOL
    Kr = jnp.asarray(_smooth_matrix(F), dtype=jnp.bfloat16)
    Kc = jnp.asarray(_smooth_matrix(T).T, dtype=jnp.bfloat16)
    Pm = jnp.asarray(
        (np.arange(T)[:, None] // _POOL == np.arange(nT)[None, :]),
        dtype=jnp.bfloat16)
    bb = 2 if B % 2 == 0 else 1
    res = pl.pallas_call(
        _hog_body,
        grid=(B // bb,),
        in_specs=[
            pl.BlockSpec((bb, 1, F, T), lambda i: (i, 0, 0, 0)),
            pl.BlockSpec((F, F), lambda i: (0, 0)),
            pl.BlockSpec((T, T), lambda i: (0, 0)),
            pl.BlockSpec((T, nT), lambda i: (0, 0)),
        ],
        out_specs=pl.BlockSpec((bb, _NBINS, nF, nT), lambda i: (i, 0, 0, 0)),
        out_shape=jax.ShapeDtypeStruct((B, _NBINS, nF, nT), jnp.float32),
        compiler_params=pltpu.CompilerParams(dimension_semantics=("arbitrary",)),
    )(x, Kr, Kc, Pm)
    return jnp.transpose(res, (0, 2, 3, 1)).reshape(B, nF * nT, _NBINS)


# speedup vs baseline: 1.0044x; 1.0044x over previous
import math

import jax
import jax.numpy as jnp
import numpy as np
from jax import lax
from jax.experimental import pallas as pl
from jax.experimental.pallas import tpu as pltpu

_NBINS = 9
_POOL = 8
_GW = 16

_W1D = np.exp(-0.5 * ((np.arange(_GW, dtype=np.float64) - (_GW - 1) / 2.0) / (_GW // 2)) ** 2)
_GNORM = float(1.0 / (_W1D.sum() ** 2))
_COTS = [float(1.0 / np.tan(k * np.pi / _NBINS)) for k in range(1, _NBINS)]


def _smooth_matrix(n: int) -> np.ndarray:
    m = np.zeros((n, n), dtype=np.float64)
    for i in range(n):
        for d in (i - 1, i, i + 1):
            src = abs(d) if d < 0 else (2 * n - 2 - d if d >= n else d)
            m[i, src] += 2.0 if d == i else 1.0
    return m


def _hog_body(x_ref, kr_ref, kc_ref, p_ref, o_ref):
    Xb = x_ref[0, 0].astype(jnp.bfloat16)
    F, T = Xb.shape
    nF = F // _POOL
    nT = T // _POOL

    sv = lax.dot_general(kr_ref[...], Xb, (((1,), (0,)), ((), ())),
                         preferred_element_type=jnp.float32)
    sh = lax.dot_general(Xb, kc_ref[...], (((1,), (0,)), ((), ())),
                         preferred_element_type=jnp.float32)

    svl = jnp.concatenate([sv[:, 1:2], sv[:, :-1]], axis=1)
    svr = jnp.concatenate([sv[:, 1:], sv[:, T - 2:T - 1]], axis=1)
    gx = svl - svr
    shu = jnp.concatenate([sh[1:2, :], sh[:-1, :]], axis=0)
    shd = jnp.concatenate([sh[1:, :], sh[F - 2:F - 1, :]], axis=0)
    gy = shu - shd

    mag = jnp.sqrt(gx * gx + gy * gy)

    fi = lax.broadcasted_iota(jnp.int32, (F, 1), 0) % _GW
    fj = lax.broadcasted_iota(jnp.int32, (1, T), 1) % _GW
    half = jnp.float32((_GW - 1) / 2.0)
    inv_std = jnp.float32(1.0 / (_GW // 2))
    wi = jnp.exp(-0.5 * ((fi.astype(jnp.float32) - half) * inv_std) ** 2)
    wj = jnp.exp(-0.5 * ((fj.astype(jnp.float32) - half) * inv_std) ** 2)
    wmag = mag * (wi * jnp.float32(_GNORM)) * wj

    u = jnp.abs(gx)
    v = jnp.where(gx < 0, -gy, gy)
    r = jnp.where(gx == 0.0, jnp.float32(np.inf), v / u)

    wmag_bf = wmag.astype(jnp.bfloat16)
    P = p_ref[...]
    pooled = []
    for k in range(_NBINS):
        Sk = wmag_bf if k == 0 else jnp.where(
            r <= jnp.float32(_COTS[k - 1]), wmag_bf, jnp.bfloat16(0.0))
        qk = lax.dot_general(Sk, P, (((1,), (0,)), ((), ())),
                             preferred_element_type=jnp.float32)
        pooled.append(qk.reshape(nF, _POOL, nT).sum(axis=1))

    hist = [pooled[k] - pooled[k + 1] for k in range(_NBINS - 1)] + [pooled[_NBINS - 1]]
    ss = hist[0] * hist[0]
    for k in range(1, _NBINS):
        ss = ss + hist[k] * hist[k]
    inv = 1.0 / jnp.maximum(jnp.sqrt(ss), 1e-12)
    for k in range(_NBINS):
        o_ref[0, k] = hist[k] * inv


def kernel(x):
    B, C, F, T = x.shape
    nF = F // _POOL
    nT = T // _POOL
    Kr = jnp.asarray(_smooth_matrix(F), dtype=jnp.bfloat16)
    Kc = jnp.asarray(_smooth_matrix(T).T, dtype=jnp.bfloat16)
    Pm = jnp.asarray(
        (np.arange(T)[:, None] // _POOL == np.arange(nT)[None, :]),
        dtype=jnp.bfloat16)
    res = pl.pallas_call(
        _hog_body,
        grid=(B,),
        in_specs=[
            pl.BlockSpec((1, 1, F, T), lambda i: (i, 0, 0, 0)),
            pl.BlockSpec((F, F), lambda i: (0, 0)),
            pl.BlockSpec((T, T), lambda i: (0, 0)),
            pl.BlockSpec((T, nT), lambda i: (0, 0)),
        ],
        out_specs=pl.BlockSpec((1, _NBINS, nF, nT), lambda i: (i, 0, 0, 0)),
        out_shape=jax.ShapeDtypeStruct((B, _NBINS, nF, nT), jnp.float32),
        compiler_params=pltpu.CompilerParams(dimension_semantics=("arbitrary",)),
    )(x, Kr, Kc, Pm)
    return jnp.transpose(res, (0, 2, 3, 1)).reshape(B, nF * nT, _NBINS)


# speedup vs baseline: 1.0257x; 1.0212x over previous
import math

import jax
import jax.numpy as jnp
import numpy as np
from jax import lax
from jax.experimental import pallas as pl
from jax.experimental.pallas import tpu as pltpu

_NBINS = 9
_POOL = 8
_GW = 16

_W1D = np.exp(-0.5 * ((np.arange(_GW, dtype=np.float64) - (_GW - 1) / 2.0) / (_GW // 2)) ** 2)
_GNORM = float(1.0 / (_W1D.sum() ** 2))
_COTS = [float(1.0 / np.tan(k * np.pi / _NBINS)) for k in range(1, _NBINS)]


def _smooth_matrix(n: int) -> np.ndarray:
    m = np.zeros((n, n), dtype=np.float64)
    for i in range(n):
        for d in (i - 1, i, i + 1):
            src = abs(d) if d < 0 else (2 * n - 2 - d if d >= n else d)
            m[i, src] += 2.0 if d == i else 1.0
    return m


def _hog_body(x_ref, kr_ref, kc_ref, p_ref, o_ref):
    Xb = x_ref[0, 0].astype(jnp.bfloat16)
    F, T = Xb.shape
    nF = F // _POOL
    nT = T // _POOL

    sv = lax.dot_general(kr_ref[...], Xb, (((1,), (0,)), ((), ())),
                         preferred_element_type=jnp.float32)
    sh = lax.dot_general(Xb, kc_ref[...], (((1,), (0,)), ((), ())),
                         preferred_element_type=jnp.float32)

    svl = jnp.concatenate([sv[:, 1:2], sv[:, :-1]], axis=1)
    svr = jnp.concatenate([sv[:, 1:], sv[:, T - 2:T - 1]], axis=1)
    gx = svl - svr
    shu = jnp.concatenate([sh[1:2, :], sh[:-1, :]], axis=0)
    shd = jnp.concatenate([sh[1:, :], sh[F - 2:F - 1, :]], axis=0)
    gy = shu - shd

    mag = jnp.sqrt(gx * gx + gy * gy)

    fi = lax.broadcasted_iota(jnp.int32, (F, 1), 0) % _GW
    fj = lax.broadcasted_iota(jnp.int32, (1, T), 1) % _GW
    half = jnp.float32((_GW - 1) / 2.0)
    inv_std = jnp.float32(1.0 / (_GW // 2))
    wi = jnp.exp(-0.5 * ((fi.astype(jnp.float32) - half) * inv_std) ** 2)
    wj = jnp.exp(-0.5 * ((fj.astype(jnp.float32) - half) * inv_std) ** 2)
    wmag = mag * (wi * jnp.float32(_GNORM)) * wj

    u = jnp.abs(gx)
    v = jnp.where(gx < 0, -gy, gy)
    r = jnp.where(gx == 0.0, jnp.float32(np.inf), v / u)

    wmag_bf = wmag.astype(jnp.bfloat16)
    P = p_ref[...]
    pooled = []
    for k in range(_NBINS):
        Sk = wmag_bf if k == 0 else jnp.where(
            r <= jnp.float32(_COTS[k - 1]), wmag_bf, jnp.bfloat16(0.0))
        qk = lax.dot_general(Sk, P, (((1,), (0,)), ((), ())),
                             preferred_element_type=jnp.float32)
        pooled.append(qk.reshape(nF, _POOL, nT).sum(axis=1))

    hist = [pooled[k] - pooled[k + 1] for k in range(_NBINS - 1)] + [pooled[_NBINS - 1]]
    ss = hist[0] * hist[0]
    for k in range(1, _NBINS):
        ss = ss + hist[k] * hist[k]
    inv = 1.0 / jnp.maximum(jnp.sqrt(ss), 1e-12)
    for k in range(_NBINS):
        o_ref[k, 0] = hist[k] * inv


def kernel(x):
    B, C, F, T = x.shape
    nF = F // _POOL
    nT = T // _POOL
    Kr = jnp.asarray(_smooth_matrix(F), dtype=jnp.bfloat16)
    Kc = jnp.asarray(_smooth_matrix(T).T, dtype=jnp.bfloat16)
    Pm = jnp.asarray(
        (np.arange(T)[:, None] // _POOL == np.arange(nT)[None, :]),
        dtype=jnp.bfloat16)
    res = pl.pallas_call(
        _hog_body,
        grid=(B,),
        in_specs=[
            pl.BlockSpec((1, 1, F, T), lambda i: (i, 0, 0, 0)),
            pl.BlockSpec((F, F), lambda i: (0, 0)),
            pl.BlockSpec((T, T), lambda i: (0, 0)),
            pl.BlockSpec((T, nT), lambda i: (0, 0)),
        ],
        out_specs=pl.BlockSpec((_NBINS, 1, nF, nT), lambda i: (0, i, 0, 0)),
        out_shape=jax.ShapeDtypeStruct((_NBINS, B, nF, nT), jnp.float32),
        compiler_params=pltpu.CompilerParams(dimension_semantics=("arbitrary",)),
    )(x, Kr, Kc, Pm)
    return jnp.transpose(res.reshape(_NBINS, B, nF * nT), (1, 2, 0))
